# baseline (speedup 1.0000x reference)
import functools

import jax
import jax.numpy as jnp
from jax import lax
from jax.experimental import pallas as pl
from jax.experimental.pallas import tpu as pltpu
from jax.experimental.pallas import tpu_sc as plsc

VOCAB = 100000
D = 1024
B = 4
S = 4096
EPS = 1e-5
SCALE = 32.0

L = 16
NC = 2
NS = 16
NW = NC * NS
S_PER_W = S // NW
CH = 32
NCHUNK = S_PER_W // CH
NSLICE = D // L
U = 4

_INV_D = 1.0 / D


def _layernorm_row(rows_v, pos_v, wv, bv, r):
    def p1(jj, carry):
        acc, acc2 = carry
        for u in range(U):
            sl = pl.ds((jj * U + u) * L, L)
            x = rows_v[r, sl] * SCALE + pos_v[r, sl]
            rows_v[r, sl] = x
            acc = acc + x
            acc2 = acc2 + x * x
        return acc, acc2

    zero = jnp.zeros((L,), jnp.float32)
    acc, acc2 = lax.fori_loop(0, NSLICE // U, p1, (zero, zero))

    mean = jnp.sum(acc) * _INV_D
    var = jnp.sum(acc2) * _INV_D - mean * mean

    vb = jnp.full((L,), var + EPS, jnp.float32)
    ib = plsc.bitcast(vb, jnp.int32)
    y = plsc.bitcast(jnp.int32(0x5F3759DF) - (ib >> 1), jnp.float32)
    for _ in range(3):
        y = y * (1.5 - 0.5 * vb * y * y)
    inv = y
    mean_v = jnp.full((L,), mean, jnp.float32)

    def p2(jj, _):
        for u in range(U):
            sl = pl.ds((jj * U + u) * L, L)
            x = rows_v[r, sl]
            rows_v[r, sl] = (x - mean_v) * inv * wv[sl] + bv[sl]
        return 0

    lax.fori_loop(0, NSLICE // U, p2, 0)


@functools.partial(
    pl.kernel,
    mesh=plsc.VectorSubcoreMesh(core_axis_name="c", subcore_axis_name="s"),
    out_type=jax.ShapeDtypeStruct((B * S, D), jnp.float32),
    compiler_params=pltpu.CompilerParams(needs_layout_passes=False),
    scratch_types=[
        pltpu.VMEM((CH,), jnp.int32),
        pltpu.VMEM((CH, D), jnp.float32),
        pltpu.VMEM((CH, D), jnp.float32),
        pltpu.VMEM((D,), jnp.float32),
        pltpu.VMEM((D,), jnp.float32),
        pltpu.SemaphoreType.DMA,
    ],
)
def _emb_frontend(seqs_hbm, table_hbm, pos_hbm, w_hbm, b_hbm, out_hbm,
                  idx_v, pos_v, rows_v, wv, bv, sem):
    wid = lax.axis_index("s") * NC + lax.axis_index("c")
    s_base = wid * S_PER_W
    pltpu.sync_copy(w_hbm, wv)
    pltpu.sync_copy(b_hbm, bv)

    def chunk_body(c, _):
        s0 = s_base + c * CH
        pltpu.sync_copy(pos_hbm.at[pl.ds(s0, CH)], pos_v)

        def batch_body(b, _):
            row0 = b * S + s0
            pltpu.sync_copy(seqs_hbm.at[pl.ds(row0, CH)], idx_v)
            pltpu.async_copy(table_hbm.at[idx_v], rows_v, sem).wait()

            def row_body(r, _):
                _layernorm_row(rows_v, pos_v, wv, bv, r)
                return 0

            lax.fori_loop(0, CH, row_body, 0)
            pltpu.sync_copy(rows_v, out_hbm.at[pl.ds(row0, CH)])
            return 0

        lax.fori_loop(0, B, batch_body, 0)
        return 0

    lax.fori_loop(0, NCHUNK, chunk_body, 0)


def kernel(seqs, padding_mask, table, ln_weight, ln_bias, pos_enc):
    out = _emb_frontend(seqs.reshape(B * S), table, pos_enc, ln_weight,
                        ln_bias)
    return out.reshape(B, S, D), padding_mask
